# baseline (speedup 1.0000x reference)
import functools

import jax
import jax.numpy as jnp
from jax import lax
from jax.experimental import pallas as pl
from jax.experimental.pallas import tpu as pltpu
from jax.experimental.pallas import tpu_sc as plsc

B = 2
S = 2048
N = B * S
H = 1024
I = 1024
I2 = 2 * I
E = 8
ALPHA = 1.702
LIMIT = 7.0

BLK = 128
NBLK = 72
ND = NBLK * BLK

NC = 2
NS = 16
NW = NC * NS
TPW = N // NW
CH = 32
NCH = TPW // CH

TB = 512


def _router_body(x_ref, gw_ref, gb_ref, e0_ref, e1_ref, w0_ref, w1_ref):
    xb = x_ref[...]
    logits = lax.dot_general(xb, gw_ref[...], (((1,), (1,)), ((), ())),
                             preferred_element_type=jnp.float32)
    logits = logits + gb_ref[...][0:1, :]
    ids = lax.broadcasted_iota(jnp.int32, logits.shape, 1)
    m1 = jnp.max(logits, axis=1, keepdims=True)
    a1 = jnp.min(jnp.where(logits == m1, ids, E), axis=1, keepdims=True)
    masked = jnp.where(ids == a1, -jnp.inf, logits)
    m2 = jnp.max(masked, axis=1, keepdims=True)
    a2 = jnp.min(jnp.where(masked == m2, ids, E), axis=1, keepdims=True)
    t = jnp.exp(m2 - m1)
    e0_ref[...] = a1
    e1_ref[...] = a2
    w0_ref[...] = jnp.broadcast_to(1.0 / (1.0 + t), (TB, 16))
    w1_ref[...] = jnp.broadcast_to(t / (1.0 + t), (TB, 16))


def _router(x, gate_w, gate_b2d):
    return pl.pallas_call(
        _router_body,
        grid=(N // TB,),
        in_specs=[
            pl.BlockSpec((TB, H), lambda i: (i, 0)),
            pl.BlockSpec((E, H), lambda i: (0, 0)),
            pl.BlockSpec((E, E), lambda i: (0, 0)),
        ],
        out_specs=[
            pl.BlockSpec((TB, 1), lambda i: (i, 0)),
            pl.BlockSpec((TB, 1), lambda i: (i, 0)),
            pl.BlockSpec((TB, 16), lambda i: (i, 0)),
            pl.BlockSpec((TB, 16), lambda i: (i, 0)),
        ],
        out_shape=[
            jax.ShapeDtypeStruct((N, 1), jnp.int32),
            jax.ShapeDtypeStruct((N, 1), jnp.int32),
            jax.ShapeDtypeStruct((N, 16), jnp.float32),
            jax.ShapeDtypeStruct((N, 16), jnp.float32),
        ],
    )(x, gate_w, gate_b2d)


def _perm_body(ef_ref, pos_ref, bid_ref):
    ef = ef_ref[...]
    rr = lax.broadcasted_iota(jnp.int32, (128, 128), 0)
    cc = lax.broadcasted_iota(jnp.int32, (128, 128), 1)
    tri = (rr <= cc).astype(jnp.float32)
    r64 = lax.broadcasted_iota(jnp.int32, (64, 64), 0)
    c64 = lax.broadcasted_iota(jnp.int32, (64, 64), 1)
    lstrict = (c64 < r64).astype(jnp.float32)
    lane = lax.broadcasted_iota(jnp.int32, (1, 128), 1).astype(jnp.float32)

    rank = jnp.zeros((64, 128), jnp.float32)
    base_sel = jnp.zeros((64, 128), jnp.float32)
    bid = jnp.zeros((1, 128), jnp.float32)
    bstart = jnp.float32(0.0)
    for e in range(E):
        xe = (ef == e).astype(jnp.float32)
        cum_inc = lax.dot_general(xe, tri, (((1,), (0,)), ((), ())),
                                  preferred_element_type=jnp.float32)
        rs = jnp.sum(xe, axis=1, keepdims=True)
        rowpref = lax.dot_general(lstrict, rs, (((1,), (0,)), ((), ())),
                                  preferred_element_type=jnp.float32)
        rank = rank + (cum_inc - xe + rowpref) * xe
        cnt = jnp.sum(rs)
        base_sel = base_sel + (bstart * BLK) * xe
        bstart = bstart + jnp.ceil(cnt / BLK)
        bid = bid + (lane >= bstart).astype(jnp.float32)
    pos_ref[...] = (base_sel + rank).astype(jnp.int32)
    bid_ref[...] = jnp.minimum(bid, E - 1).astype(jnp.int32)


def _perm(eflat):
    return pl.pallas_call(
        _perm_body,
        out_shape=[
            jax.ShapeDtypeStruct((64, 128), jnp.int32),
            jax.ShapeDtypeStruct((1, 128), jnp.int32),
        ],
    )(eflat)


@functools.cache
def _make_dispatch():
    mesh = plsc.VectorSubcoreMesh(core_axis_name="c", subcore_axis_name="s")

    @functools.partial(
        pl.kernel,
        mesh=mesh,
        out_type=jax.ShapeDtypeStruct((ND, H), jnp.float32),
        scratch_types=[
            pltpu.VMEM((CH, H), jnp.float32),
            pltpu.VMEM((NCH, CH), jnp.int32),
            pltpu.VMEM((NCH, CH), jnp.int32),
            pltpu.SemaphoreType.DMA,
            pltpu.SemaphoreType.DMA,
        ],
    )
    def _dispatch_k(x_hbm, p0_hbm, p1_hbm, xd_hbm, rows_v, i0_v, i1_v, s0, s1):
        wid = lax.axis_index("s") * NC + lax.axis_index("c")
        pltpu.sync_copy(p0_hbm.at[wid], i0_v)
        pltpu.sync_copy(p1_hbm.at[wid], i1_v)
        for j in range(NCH):
            base = wid * TPW + j * CH
            pltpu.sync_copy(x_hbm.at[pl.ds(base, CH)], rows_v)
            c0 = pltpu.async_copy(rows_v, xd_hbm.at[i0_v.at[j]], s0)
            c1 = pltpu.async_copy(rows_v, xd_hbm.at[i1_v.at[j]], s1)
            c0.wait()
            c1.wait()

    return _dispatch_k


def _prep_body(w4_ref, wg_ref, wl_ref, sg, sl):
    for e in range(E):
        pltpu.make_async_copy(w4_ref.at[e, :, 0, :], wg_ref.at[e], sg).start()
        pltpu.make_async_copy(w4_ref.at[e, :, 1, :], wl_ref.at[e], sl).start()
    for e in range(E):
        pltpu.make_async_copy(w4_ref.at[e, :, 0, :], wg_ref.at[e], sg).wait()
        pltpu.make_async_copy(w4_ref.at[e, :, 1, :], wl_ref.at[e], sl).wait()


def _prep(fc1_w4):
    return pl.pallas_call(
        _prep_body,
        in_specs=[pl.BlockSpec(memory_space=pl.ANY)],
        out_specs=[pl.BlockSpec(memory_space=pl.ANY),
                   pl.BlockSpec(memory_space=pl.ANY)],
        out_shape=[jax.ShapeDtypeStruct((E, I, H), jnp.float32),
                   jax.ShapeDtypeStruct((E, I, H), jnp.float32)],
        scratch_shapes=[pltpu.SemaphoreType.DMA, pltpu.SemaphoreType.DMA],
    )(fc1_w4)


def _ffn_body(be_ref, xd_ref, wg_ref, wl_ref, bg_ref, bl_ref, w2_ref, b2_ref,
              o_ref):
    xb = xd_ref[...]
    hg = lax.dot_general(xb, wg_ref[0], (((1,), (1,)), ((), ())),
                         preferred_element_type=jnp.float32) + bg_ref[0]
    hl = lax.dot_general(xb, wl_ref[0], (((1,), (1,)), ((), ())),
                         preferred_element_type=jnp.float32) + bl_ref[0]
    hg = jnp.minimum(hg, LIMIT)
    hl = jnp.clip(hl, -LIMIT, LIMIT)
    y = hg * (1.0 / (1.0 + jnp.exp(-ALPHA * hg))) * (hl + 1.0)
    out = lax.dot_general(y, w2_ref[0], (((1,), (1,)), ((), ())),
                          preferred_element_type=jnp.float32)
    o_ref[...] = out + b2_ref[0]


def _ffn(be, xd, wg, wl, bg, bl, fc2_w, b2):
    return pl.pallas_call(
        _ffn_body,
        grid_spec=pltpu.PrefetchScalarGridSpec(
            num_scalar_prefetch=1,
            grid=(NBLK,),
            in_specs=[
                pl.BlockSpec((BLK, H), lambda i, be_r: (i, 0)),
                pl.BlockSpec((1, I, H), lambda i, be_r: (be_r[i], 0, 0)),
                pl.BlockSpec((1, I, H), lambda i, be_r: (be_r[i], 0, 0)),
                pl.BlockSpec((1, 1, I), lambda i, be_r: (be_r[i], 0, 0)),
                pl.BlockSpec((1, 1, I), lambda i, be_r: (be_r[i], 0, 0)),
                pl.BlockSpec((1, H, I), lambda i, be_r: (be_r[i], 0, 0)),
                pl.BlockSpec((1, 1, H), lambda i, be_r: (be_r[i], 0, 0)),
            ],
            out_specs=pl.BlockSpec((BLK, H), lambda i, be_r: (i, 0)),
        ),
        out_shape=jax.ShapeDtypeStruct((ND, H), jnp.float32),
    )(be, xd, wg, wl, bg, bl, fc2_w, b2)


@functools.cache
def _make_combine():
    mesh = plsc.VectorSubcoreMesh(core_axis_name="c", subcore_axis_name="s")

    @functools.partial(
        pl.kernel,
        mesh=mesh,
        out_type=jax.ShapeDtypeStruct((N, H), jnp.float32),
        scratch_types=[
            pltpu.VMEM((CH, H), jnp.float32),
            pltpu.VMEM((CH, H), jnp.float32),
            pltpu.VMEM((NCH, CH), jnp.int32),
            pltpu.VMEM((NCH, CH), jnp.int32),
            pltpu.VMEM((TPW, 16), jnp.float32),
            pltpu.VMEM((TPW, 16), jnp.float32),
            pltpu.SemaphoreType.DMA,
            pltpu.SemaphoreType.DMA,
        ],
    )
    def _combine_k(od_hbm, p0_hbm, p1_hbm, w0_hbm, w1_hbm, out_hbm,
                   a_v, b_v, i0_v, i1_v, wa_v, wb_v, sa, sb):
        wid = lax.axis_index("s") * NC + lax.axis_index("c")
        pltpu.sync_copy(p0_hbm.at[wid], i0_v)
        pltpu.sync_copy(p1_hbm.at[wid], i1_v)
        pltpu.sync_copy(w0_hbm.at[wid], wa_v)
        pltpu.sync_copy(w1_hbm.at[wid], wb_v)
        for j in range(NCH):
            ca = pltpu.async_copy(od_hbm.at[i0_v.at[j]], a_v, sa)
            cb = pltpu.async_copy(od_hbm.at[i1_v.at[j]], b_v, sb)
            ca.wait()
            cb.wait()

            def body_m(m, carry):
                wa = wa_v[j * CH + m, :]
                wb = wb_v[j * CH + m, :]
                for c in range(H // 16):
                    av = a_v[m, pl.ds(c * 16, 16)]
                    bv = b_v[m, pl.ds(c * 16, 16)]
                    a_v[m, pl.ds(c * 16, 16)] = av * wa + bv * wb
                return carry

            lax.fori_loop(0, CH, body_m, 0)
            pltpu.sync_copy(a_v, out_hbm.at[pl.ds(wid * TPW + j * CH, CH)])

    return _combine_k


def kernel(hidden_states, gate_w, gate_b, fc1_w, fc1_b, fc2_w, fc2_b):
    x = hidden_states.reshape(N, H)
    gb2 = jnp.broadcast_to(gate_b[None, :], (E, E))
    e0, e1, w0, w1 = _router(x, gate_w, gb2)

    eflat = jnp.concatenate([e0, e1], axis=1).reshape(64, 128)
    pos2d, bid = _perm(eflat)
    pos = pos2d.reshape(N, 2)
    p0 = pos[:, 0].reshape(NW, NCH, CH)
    p1 = pos[:, 1].reshape(NW, NCH, CH)
    be = bid.reshape(128)[:NBLK]

    xd = _make_dispatch()(x, p0, p1)

    bg = fc1_b[:, 0::2].reshape(E, 1, I)
    bl = fc1_b[:, 1::2].reshape(E, 1, I)
    fc1_w4 = fc1_w.reshape(E, I, 2, H)
    wg, wl = _prep(fc1_w4)
    od = _ffn(be, xd, wg, wl, bg, bl, fc2_w, fc2_b.reshape(E, 1, H))

    out = _make_combine()(od, p0, p1,
                          w0.reshape(NW, TPW, 16), w1.reshape(NW, TPW, 16))
    return out.reshape(B, S, H)


# speedup vs baseline: 5.5217x; 5.5217x over previous
import functools

import jax
import jax.numpy as jnp
from jax import lax
from jax.experimental import pallas as pl
from jax.experimental.pallas import tpu as pltpu
from jax.experimental.pallas import tpu_sc as plsc

B = 2
S = 2048
N = B * S
H = 1024
I = 1024
I2 = 2 * I
E = 8
ALPHA = 1.702
LIMIT = 7.0

BLK = 128
NBLK = 72
ND = NBLK * BLK

NC = 2
NS = 16
NW = NC * NS
TPW = N // NW
CH = 32
NCH = TPW // CH

TB = 512


def _router_body(x_ref, gw_ref, gb_ref, e0_ref, e1_ref, w0_ref, w1_ref):
    xb = x_ref[...]
    logits = lax.dot_general(xb, gw_ref[...], (((1,), (1,)), ((), ())),
                             preferred_element_type=jnp.float32)
    logits = logits + gb_ref[...][0:1, :]
    ids = lax.broadcasted_iota(jnp.int32, logits.shape, 1)
    m1 = jnp.max(logits, axis=1, keepdims=True)
    a1 = jnp.min(jnp.where(logits == m1, ids, E), axis=1, keepdims=True)
    masked = jnp.where(ids == a1, -jnp.inf, logits)
    m2 = jnp.max(masked, axis=1, keepdims=True)
    a2 = jnp.min(jnp.where(masked == m2, ids, E), axis=1, keepdims=True)
    t = jnp.exp(m2 - m1)
    e0_ref[...] = a1
    e1_ref[...] = a2
    w0_ref[...] = jnp.broadcast_to(1.0 / (1.0 + t), (TB, 16))
    w1_ref[...] = jnp.broadcast_to(t / (1.0 + t), (TB, 16))


def _router(x, gate_w, gate_b2d):
    return pl.pallas_call(
        _router_body,
        grid=(N // TB,),
        in_specs=[
            pl.BlockSpec((TB, H), lambda i: (i, 0)),
            pl.BlockSpec((E, H), lambda i: (0, 0)),
            pl.BlockSpec((E, E), lambda i: (0, 0)),
        ],
        out_specs=[
            pl.BlockSpec((TB, 1), lambda i: (i, 0)),
            pl.BlockSpec((TB, 1), lambda i: (i, 0)),
            pl.BlockSpec((TB, 16), lambda i: (i, 0)),
            pl.BlockSpec((TB, 16), lambda i: (i, 0)),
        ],
        out_shape=[
            jax.ShapeDtypeStruct((N, 1), jnp.int32),
            jax.ShapeDtypeStruct((N, 1), jnp.int32),
            jax.ShapeDtypeStruct((N, 16), jnp.float32),
            jax.ShapeDtypeStruct((N, 16), jnp.float32),
        ],
    )(x, gate_w, gate_b2d)


def _perm_body(ef_ref, pos_ref, bid_ref):
    ef = ef_ref[...]
    rr = lax.broadcasted_iota(jnp.int32, (128, 128), 0)
    cc = lax.broadcasted_iota(jnp.int32, (128, 128), 1)
    tri = (rr <= cc).astype(jnp.float32)
    r64 = lax.broadcasted_iota(jnp.int32, (64, 64), 0)
    c64 = lax.broadcasted_iota(jnp.int32, (64, 64), 1)
    lstrict = (c64 < r64).astype(jnp.float32)
    lane = lax.broadcasted_iota(jnp.int32, (1, 128), 1).astype(jnp.float32)

    rank = jnp.zeros((64, 128), jnp.float32)
    base_sel = jnp.zeros((64, 128), jnp.float32)
    bid = jnp.zeros((1, 128), jnp.float32)
    bstart = jnp.float32(0.0)
    for e in range(E):
        xe = (ef == e).astype(jnp.float32)
        cum_inc = lax.dot_general(xe, tri, (((1,), (0,)), ((), ())),
                                  preferred_element_type=jnp.float32)
        rs = jnp.sum(xe, axis=1, keepdims=True)
        rowpref = lax.dot_general(lstrict, rs, (((1,), (0,)), ((), ())),
                                  preferred_element_type=jnp.float32)
        rank = rank + (cum_inc - xe + rowpref) * xe
        cnt = jnp.sum(rs)
        base_sel = base_sel + (bstart * BLK) * xe
        bstart = bstart + jnp.ceil(cnt / BLK)
        bid = bid + (lane >= bstart).astype(jnp.float32)
    pos_ref[...] = (base_sel + rank).astype(jnp.int32)
    bid_ref[...] = jnp.minimum(bid, E - 1).astype(jnp.int32)


def _perm(eflat):
    return pl.pallas_call(
        _perm_body,
        out_shape=[
            jax.ShapeDtypeStruct((64, 128), jnp.int32),
            jax.ShapeDtypeStruct((1, 128), jnp.int32),
        ],
    )(eflat)


@functools.cache
def _make_dispatch():
    mesh = plsc.VectorSubcoreMesh(core_axis_name="c", subcore_axis_name="s")

    @functools.partial(
        pl.kernel,
        mesh=mesh,
        out_type=jax.ShapeDtypeStruct((ND, H), jnp.float32),
        scratch_types=[
            pltpu.VMEM((CH, H), jnp.float32),
            pltpu.VMEM((NCH, CH), jnp.int32),
            pltpu.VMEM((NCH, CH), jnp.int32),
            pltpu.SemaphoreType.DMA,
            pltpu.SemaphoreType.DMA,
        ],
    )
    def _dispatch_k(x_hbm, p0_hbm, p1_hbm, xd_hbm, rows_v, i0_v, i1_v, s0, s1):
        wid = lax.axis_index("s") * NC + lax.axis_index("c")
        pltpu.sync_copy(p0_hbm.at[wid], i0_v)
        pltpu.sync_copy(p1_hbm.at[wid], i1_v)
        for j in range(NCH):
            base = wid * TPW + j * CH
            pltpu.sync_copy(x_hbm.at[pl.ds(base, CH)], rows_v)
            c0 = pltpu.async_copy(rows_v, xd_hbm.at[i0_v.at[j]], s0)
            c1 = pltpu.async_copy(rows_v, xd_hbm.at[i1_v.at[j]], s1)
            c0.wait()
            c1.wait()

    return _dispatch_k


def _ffn_body(be_ref, xd_ref, wg_ref, wl_ref, bg_ref, bl_ref, w2_ref, b2_ref,
              o_ref):
    xb = xd_ref[...]
    hg = lax.dot_general(xb, wg_ref[0], (((1,), (1,)), ((), ())),
                         preferred_element_type=jnp.float32) + bg_ref[0]
    hl = lax.dot_general(xb, wl_ref[0], (((1,), (1,)), ((), ())),
                         preferred_element_type=jnp.float32) + bl_ref[0]
    hg = jnp.minimum(hg, LIMIT)
    hl = jnp.clip(hl, -LIMIT, LIMIT)
    y = hg * (1.0 / (1.0 + jnp.exp(-ALPHA * hg))) * (hl + 1.0)
    out = lax.dot_general(y, w2_ref[0], (((1,), (1,)), ((), ())),
                          preferred_element_type=jnp.float32)
    o_ref[...] = out + b2_ref[0]


def _ffn(be, xd, wg, wl, bg, bl, fc2_w, b2):
    return pl.pallas_call(
        _ffn_body,
        grid_spec=pltpu.PrefetchScalarGridSpec(
            num_scalar_prefetch=1,
            grid=(NBLK,),
            in_specs=[
                pl.BlockSpec((BLK, H), lambda i, be_r: (i, 0)),
                pl.BlockSpec((1, I, H), lambda i, be_r: (be_r[i], 0, 0)),
                pl.BlockSpec((1, I, H), lambda i, be_r: (be_r[i], 0, 0)),
                pl.BlockSpec((1, 1, I), lambda i, be_r: (be_r[i], 0, 0)),
                pl.BlockSpec((1, 1, I), lambda i, be_r: (be_r[i], 0, 0)),
                pl.BlockSpec((1, H, I), lambda i, be_r: (be_r[i], 0, 0)),
                pl.BlockSpec((1, 1, H), lambda i, be_r: (be_r[i], 0, 0)),
            ],
            out_specs=pl.BlockSpec((BLK, H), lambda i, be_r: (i, 0)),
        ),
        out_shape=jax.ShapeDtypeStruct((ND, H), jnp.float32),
    )(be, xd, wg, wl, bg, bl, fc2_w, b2)


@functools.cache
def _make_combine():
    mesh = plsc.VectorSubcoreMesh(core_axis_name="c", subcore_axis_name="s")

    @functools.partial(
        pl.kernel,
        mesh=mesh,
        out_type=jax.ShapeDtypeStruct((N, H), jnp.float32),
        scratch_types=[
            pltpu.VMEM((CH, H), jnp.float32),
            pltpu.VMEM((CH, H), jnp.float32),
            pltpu.VMEM((NCH, CH), jnp.int32),
            pltpu.VMEM((NCH, CH), jnp.int32),
            pltpu.VMEM((TPW, 16), jnp.float32),
            pltpu.VMEM((TPW, 16), jnp.float32),
            pltpu.SemaphoreType.DMA,
            pltpu.SemaphoreType.DMA,
        ],
    )
    def _combine_k(od_hbm, p0_hbm, p1_hbm, w0_hbm, w1_hbm, out_hbm,
                   a_v, b_v, i0_v, i1_v, wa_v, wb_v, sa, sb):
        wid = lax.axis_index("s") * NC + lax.axis_index("c")
        pltpu.sync_copy(p0_hbm.at[wid], i0_v)
        pltpu.sync_copy(p1_hbm.at[wid], i1_v)
        pltpu.sync_copy(w0_hbm.at[wid], wa_v)
        pltpu.sync_copy(w1_hbm.at[wid], wb_v)
        for j in range(NCH):
            ca = pltpu.async_copy(od_hbm.at[i0_v.at[j]], a_v, sa)
            cb = pltpu.async_copy(od_hbm.at[i1_v.at[j]], b_v, sb)
            ca.wait()
            cb.wait()

            def body_m(m, carry):
                wa = wa_v[j * CH + m, :]
                wb = wb_v[j * CH + m, :]
                for c in range(H // 16):
                    av = a_v[m, pl.ds(c * 16, 16)]
                    bv = b_v[m, pl.ds(c * 16, 16)]
                    a_v[m, pl.ds(c * 16, 16)] = av * wa + bv * wb
                return carry

            lax.fori_loop(0, CH, body_m, 0)
            pltpu.sync_copy(a_v, out_hbm.at[pl.ds(wid * TPW + j * CH, CH)])

    return _combine_k


def kernel(hidden_states, gate_w, gate_b, fc1_w, fc1_b, fc2_w, fc2_b):
    x = hidden_states.reshape(N, H)
    gb2 = jnp.broadcast_to(gate_b[None, :], (E, E))
    e0, e1, w0, w1 = _router(x, gate_w, gb2)

    eflat = jnp.concatenate([e0, e1], axis=1).reshape(64, 128)
    pos2d, bid = _perm(eflat)
    pos = pos2d.reshape(N, 2)
    p0 = pos[:, 0].reshape(NW, NCH, CH)
    p1 = pos[:, 1].reshape(NW, NCH, CH)
    be = bid.reshape(128)[:NBLK]

    xd = _make_dispatch()(x, p0, p1)

    bg = fc1_b[:, 0::2].reshape(E, 1, I)
    bl = fc1_b[:, 1::2].reshape(E, 1, I)
    w1t = fc1_w.reshape(E, I, 2, H).transpose(0, 2, 1, 3)
    od = _ffn(be, xd, w1t[:, 0], w1t[:, 1], bg, bl, fc2_w,
              fc2_b.reshape(E, 1, H))

    out = _make_combine()(od, p0, p1,
                          w0.reshape(NW, TPW, 16), w1.reshape(NW, TPW, 16))
    return out.reshape(B, S, H)


# speedup vs baseline: 5.9848x; 1.0839x over previous
import functools

import jax
import jax.numpy as jnp
from jax import lax
from jax.experimental import pallas as pl
from jax.experimental.pallas import tpu as pltpu
from jax.experimental.pallas import tpu_sc as plsc

B = 2
S = 2048
N = B * S
H = 1024
I = 1024
I2 = 2 * I
E = 8
ALPHA = 1.702
LIMIT = 7.0

BLK = 128
NBLK = 72
ND = NBLK * BLK

NC = 2
NS = 16
NW = NC * NS
TPW = N // NW
CH = 32
NCH = TPW // CH

TB = 512


def _router_body(x_ref, gw_ref, gb_ref, e0_ref, e1_ref, w0_ref, w1_ref):
    xb = x_ref[...]
    logits = lax.dot_general(xb, gw_ref[...], (((1,), (1,)), ((), ())),
                             preferred_element_type=jnp.float32)
    logits = logits + gb_ref[...][0:1, :]
    ids = lax.broadcasted_iota(jnp.int32, logits.shape, 1)
    m1 = jnp.max(logits, axis=1, keepdims=True)
    a1 = jnp.min(jnp.where(logits == m1, ids, E), axis=1, keepdims=True)
    masked = jnp.where(ids == a1, -jnp.inf, logits)
    m2 = jnp.max(masked, axis=1, keepdims=True)
    a2 = jnp.min(jnp.where(masked == m2, ids, E), axis=1, keepdims=True)
    t = jnp.exp(m2 - m1)
    e0_ref[...] = a1
    e1_ref[...] = a2
    w0_ref[...] = jnp.broadcast_to(1.0 / (1.0 + t), (TB, 16))
    w1_ref[...] = jnp.broadcast_to(t / (1.0 + t), (TB, 16))


def _router(x, gate_w, gate_b2d):
    return pl.pallas_call(
        _router_body,
        grid=(N // TB,),
        in_specs=[
            pl.BlockSpec((TB, H), lambda i: (i, 0)),
            pl.BlockSpec((E, H), lambda i: (0, 0)),
            pl.BlockSpec((E, E), lambda i: (0, 0)),
        ],
        out_specs=[
            pl.BlockSpec((TB, 1), lambda i: (i, 0)),
            pl.BlockSpec((TB, 1), lambda i: (i, 0)),
            pl.BlockSpec((TB, 16), lambda i: (i, 0)),
            pl.BlockSpec((TB, 16), lambda i: (i, 0)),
        ],
        out_shape=[
            jax.ShapeDtypeStruct((N, 1), jnp.int32),
            jax.ShapeDtypeStruct((N, 1), jnp.int32),
            jax.ShapeDtypeStruct((N, 16), jnp.float32),
            jax.ShapeDtypeStruct((N, 16), jnp.float32),
        ],
    )(x, gate_w, gate_b2d)


def _perm_body(ef_ref, pos_ref, bid_ref):
    ef = ef_ref[...]
    rr = lax.broadcasted_iota(jnp.int32, (128, 128), 0)
    cc = lax.broadcasted_iota(jnp.int32, (128, 128), 1)
    tri = (rr <= cc).astype(jnp.float32)
    r64 = lax.broadcasted_iota(jnp.int32, (64, 64), 0)
    c64 = lax.broadcasted_iota(jnp.int32, (64, 64), 1)
    lstrict = (c64 < r64).astype(jnp.float32)
    lane = lax.broadcasted_iota(jnp.int32, (1, 128), 1).astype(jnp.float32)

    rank = jnp.zeros((64, 128), jnp.float32)
    base_sel = jnp.zeros((64, 128), jnp.float32)
    bid = jnp.zeros((1, 128), jnp.float32)
    bstart = jnp.float32(0.0)
    for e in range(E):
        xe = (ef == e).astype(jnp.float32)
        cum_inc = lax.dot_general(xe, tri, (((1,), (0,)), ((), ())),
                                  preferred_element_type=jnp.float32)
        rs = jnp.sum(xe, axis=1, keepdims=True)
        rowpref = lax.dot_general(lstrict, rs, (((1,), (0,)), ((), ())),
                                  preferred_element_type=jnp.float32)
        rank = rank + (cum_inc - xe + rowpref) * xe
        cnt = jnp.sum(rs)
        base_sel = base_sel + (bstart * BLK) * xe
        bstart = bstart + jnp.ceil(cnt / BLK)
        bid = bid + (lane >= bstart).astype(jnp.float32)
    pos_ref[...] = (base_sel + rank).astype(jnp.int32)
    bid_ref[...] = jnp.minimum(bid, E - 1).astype(jnp.int32)


def _perm(eflat):
    return pl.pallas_call(
        _perm_body,
        out_shape=[
            jax.ShapeDtypeStruct((64, 128), jnp.int32),
            jax.ShapeDtypeStruct((1, 128), jnp.int32),
        ],
    )(eflat)


@functools.cache
def _make_dispatch():
    mesh = plsc.VectorSubcoreMesh(core_axis_name="c", subcore_axis_name="s")

    @functools.partial(
        pl.kernel,
        mesh=mesh,
        out_type=jax.ShapeDtypeStruct((ND, H), jnp.float32),
        scratch_types=[
            pltpu.VMEM((CH, H), jnp.float32),
            pltpu.VMEM((NCH, CH), jnp.int32),
            pltpu.VMEM((NCH, CH), jnp.int32),
            pltpu.SemaphoreType.DMA,
            pltpu.SemaphoreType.DMA,
        ],
    )
    def _dispatch_k(x_hbm, p0_hbm, p1_hbm, xd_hbm, rows_v, i0_v, i1_v, s0, s1):
        wid = lax.axis_index("s") * NC + lax.axis_index("c")
        pltpu.sync_copy(p0_hbm.at[wid], i0_v)
        pltpu.sync_copy(p1_hbm.at[wid], i1_v)
        for j in range(NCH):
            base = wid * TPW + j * CH
            pltpu.sync_copy(x_hbm.at[pl.ds(base, CH)], rows_v)
            c0 = pltpu.async_copy(rows_v, xd_hbm.at[i0_v.at[j]], s0)
            c1 = pltpu.async_copy(rows_v, xd_hbm.at[i1_v.at[j]], s1)
            c0.wait()
            c1.wait()

    return _dispatch_k


def _ffn_body(be_ref, xd_ref, wg_ref, wl_ref, bg_ref, bl_ref, w2_ref, b2_ref,
              o_ref):
    xb = xd_ref[...]
    hg = lax.dot_general(xb, wg_ref[0, 0], (((1,), (1,)), ((), ())),
                         preferred_element_type=jnp.float32) + bg_ref[0]
    hl = lax.dot_general(xb, wl_ref[0, 0], (((1,), (1,)), ((), ())),
                         preferred_element_type=jnp.float32) + bl_ref[0]
    hg = jnp.minimum(hg, LIMIT)
    hl = jnp.clip(hl, -LIMIT, LIMIT)
    y = hg * (1.0 / (1.0 + jnp.exp(-ALPHA * hg))) * (hl + 1.0)
    out = lax.dot_general(y, w2_ref[0], (((1,), (1,)), ((), ())),
                          preferred_element_type=jnp.float32)
    o_ref[...] = out + b2_ref[0]


def _ffn(be, xd, w1t, bg, bl, fc2_w, b2):
    return pl.pallas_call(
        _ffn_body,
        grid_spec=pltpu.PrefetchScalarGridSpec(
            num_scalar_prefetch=1,
            grid=(NBLK,),
            in_specs=[
                pl.BlockSpec((BLK, H), lambda i, be_r: (i, 0)),
                pl.BlockSpec((1, 1, I, H), lambda i, be_r: (be_r[i], 0, 0, 0)),
                pl.BlockSpec((1, 1, I, H), lambda i, be_r: (be_r[i], 1, 0, 0)),
                pl.BlockSpec((1, 1, I), lambda i, be_r: (be_r[i], 0, 0)),
                pl.BlockSpec((1, 1, I), lambda i, be_r: (be_r[i], 0, 0)),
                pl.BlockSpec((1, H, I), lambda i, be_r: (be_r[i], 0, 0)),
                pl.BlockSpec((1, 1, H), lambda i, be_r: (be_r[i], 0, 0)),
            ],
            out_specs=pl.BlockSpec((BLK, H), lambda i, be_r: (i, 0)),
        ),
        out_shape=jax.ShapeDtypeStruct((ND, H), jnp.float32),
    )(be, xd, w1t, w1t, bg, bl, fc2_w, b2)


@functools.cache
def _make_combine():
    mesh = plsc.VectorSubcoreMesh(core_axis_name="c", subcore_axis_name="s")

    @functools.partial(
        pl.kernel,
        mesh=mesh,
        out_type=jax.ShapeDtypeStruct((N, H), jnp.float32),
        scratch_types=[
            pltpu.VMEM((CH, H), jnp.float32),
            pltpu.VMEM((CH, H), jnp.float32),
            pltpu.VMEM((NCH, CH), jnp.int32),
            pltpu.VMEM((NCH, CH), jnp.int32),
            pltpu.VMEM((TPW, 16), jnp.float32),
            pltpu.VMEM((TPW, 16), jnp.float32),
            pltpu.SemaphoreType.DMA,
            pltpu.SemaphoreType.DMA,
        ],
    )
    def _combine_k(od_hbm, p0_hbm, p1_hbm, w0_hbm, w1_hbm, out_hbm,
                   a_v, b_v, i0_v, i1_v, wa_v, wb_v, sa, sb):
        wid = lax.axis_index("s") * NC + lax.axis_index("c")
        pltpu.sync_copy(p0_hbm.at[wid], i0_v)
        pltpu.sync_copy(p1_hbm.at[wid], i1_v)
        pltpu.sync_copy(w0_hbm.at[wid], wa_v)
        pltpu.sync_copy(w1_hbm.at[wid], wb_v)
        for j in range(NCH):
            ca = pltpu.async_copy(od_hbm.at[i0_v.at[j]], a_v, sa)
            cb = pltpu.async_copy(od_hbm.at[i1_v.at[j]], b_v, sb)
            ca.wait()
            cb.wait()

            def body_m(m, carry):
                wa = wa_v[j * CH + m, :]
                wb = wb_v[j * CH + m, :]
                for c in range(H // 16):
                    av = a_v[m, pl.ds(c * 16, 16)]
                    bv = b_v[m, pl.ds(c * 16, 16)]
                    a_v[m, pl.ds(c * 16, 16)] = av * wa + bv * wb
                return carry

            lax.fori_loop(0, CH, body_m, 0)
            pltpu.sync_copy(a_v, out_hbm.at[pl.ds(wid * TPW + j * CH, CH)])

    return _combine_k


def kernel(hidden_states, gate_w, gate_b, fc1_w, fc1_b, fc2_w, fc2_b):
    x = hidden_states.reshape(N, H)
    gb2 = jnp.broadcast_to(gate_b[None, :], (E, E))
    e0, e1, w0, w1 = _router(x, gate_w, gb2)

    eflat = jnp.concatenate([e0, e1], axis=1).reshape(64, 128)
    pos2d, bid = _perm(eflat)
    pos = pos2d.reshape(N, 2)
    p0 = pos[:, 0].reshape(NW, NCH, CH)
    p1 = pos[:, 1].reshape(NW, NCH, CH)
    be = bid.reshape(128)[:NBLK]

    xd = _make_dispatch()(x, p0, p1)

    bg = fc1_b[:, 0::2].reshape(E, 1, I)
    bl = fc1_b[:, 1::2].reshape(E, 1, I)
    w1t = fc1_w.reshape(E, I, 2, H).transpose(0, 2, 1, 3)
    od = _ffn(be, xd, w1t, bg, bl, fc2_w, fc2_b.reshape(E, 1, H))

    out = _make_combine()(od, p0, p1,
                          w0.reshape(NW, TPW, 16), w1.reshape(NW, TPW, 16))
    return out.reshape(B, S, H)


# speedup vs baseline: 6.2450x; 1.0435x over previous
import functools

import jax
import jax.numpy as jnp
from jax import lax
from jax.experimental import pallas as pl
from jax.experimental.pallas import tpu as pltpu
from jax.experimental.pallas import tpu_sc as plsc

B = 2
S = 2048
N = B * S
H = 1024
I = 1024
I2 = 2 * I
E = 8
ALPHA = 1.702
LIMIT = 7.0

BLK = 128
NBLK = 72
ND = NBLK * BLK

NC = 2
NS = 16
NW = NC * NS
TPW = N // NW
CH = 32
NCH = TPW // CH

TB = 512


def _router_body(x_ref, gw_ref, gb_ref, e0_ref, e1_ref, w0_ref, w1_ref):
    xb = x_ref[...]
    logits = lax.dot_general(xb, gw_ref[...], (((1,), (1,)), ((), ())),
                             preferred_element_type=jnp.float32)
    logits = logits + gb_ref[...][0:1, :]
    ids = lax.broadcasted_iota(jnp.int32, logits.shape, 1)
    m1 = jnp.max(logits, axis=1, keepdims=True)
    a1 = jnp.min(jnp.where(logits == m1, ids, E), axis=1, keepdims=True)
    masked = jnp.where(ids == a1, -jnp.inf, logits)
    m2 = jnp.max(masked, axis=1, keepdims=True)
    a2 = jnp.min(jnp.where(masked == m2, ids, E), axis=1, keepdims=True)
    t = jnp.exp(m2 - m1)
    e0_ref[...] = a1
    e1_ref[...] = a2
    w0_ref[...] = jnp.broadcast_to(1.0 / (1.0 + t), (TB, 16))
    w1_ref[...] = jnp.broadcast_to(t / (1.0 + t), (TB, 16))


def _router(x, gate_w, gate_b2d):
    return pl.pallas_call(
        _router_body,
        grid=(N // TB,),
        in_specs=[
            pl.BlockSpec((TB, H), lambda i: (i, 0)),
            pl.BlockSpec((E, H), lambda i: (0, 0)),
            pl.BlockSpec((E, E), lambda i: (0, 0)),
        ],
        out_specs=[
            pl.BlockSpec((TB, 1), lambda i: (i, 0)),
            pl.BlockSpec((TB, 1), lambda i: (i, 0)),
            pl.BlockSpec((TB, 16), lambda i: (i, 0)),
            pl.BlockSpec((TB, 16), lambda i: (i, 0)),
        ],
        out_shape=[
            jax.ShapeDtypeStruct((N, 1), jnp.int32),
            jax.ShapeDtypeStruct((N, 1), jnp.int32),
            jax.ShapeDtypeStruct((N, 16), jnp.float32),
            jax.ShapeDtypeStruct((N, 16), jnp.float32),
        ],
    )(x, gate_w, gate_b2d)


def _perm_body(ef_ref, pos_ref, bid_ref):
    ef = ef_ref[...]
    rr = lax.broadcasted_iota(jnp.int32, (128, 128), 0)
    cc = lax.broadcasted_iota(jnp.int32, (128, 128), 1)
    tri = (rr <= cc).astype(jnp.float32)
    r64 = lax.broadcasted_iota(jnp.int32, (64, 64), 0)
    c64 = lax.broadcasted_iota(jnp.int32, (64, 64), 1)
    lstrict = (c64 < r64).astype(jnp.float32)
    lane = lax.broadcasted_iota(jnp.int32, (1, 128), 1).astype(jnp.float32)

    rank = jnp.zeros((64, 128), jnp.float32)
    base_sel = jnp.zeros((64, 128), jnp.float32)
    bid = jnp.zeros((1, 128), jnp.float32)
    bstart = jnp.float32(0.0)
    for e in range(E):
        xe = (ef == e).astype(jnp.float32)
        cum_inc = lax.dot_general(xe, tri, (((1,), (0,)), ((), ())),
                                  preferred_element_type=jnp.float32)
        rs = jnp.sum(xe, axis=1, keepdims=True)
        rowpref = lax.dot_general(lstrict, rs, (((1,), (0,)), ((), ())),
                                  preferred_element_type=jnp.float32)
        rank = rank + (cum_inc - xe + rowpref) * xe
        cnt = jnp.sum(rs)
        base_sel = base_sel + (bstart * BLK) * xe
        bstart = bstart + jnp.ceil(cnt / BLK)
        bid = bid + (lane >= bstart).astype(jnp.float32)
    pos_ref[...] = (base_sel + rank).astype(jnp.int32)
    bid_ref[...] = jnp.minimum(bid, E - 1).astype(jnp.int32)


def _perm(eflat):
    return pl.pallas_call(
        _perm_body,
        out_shape=[
            jax.ShapeDtypeStruct((64, 128), jnp.int32),
            jax.ShapeDtypeStruct((1, 128), jnp.int32),
        ],
    )(eflat)


@functools.cache
def _make_dispatch():
    mesh = plsc.VectorSubcoreMesh(core_axis_name="c", subcore_axis_name="s")

    @functools.partial(
        pl.kernel,
        mesh=mesh,
        out_type=jax.ShapeDtypeStruct((ND, H), jnp.float32),
        scratch_types=[
            pltpu.VMEM((CH, H), jnp.float32),
            pltpu.VMEM((NCH, CH), jnp.int32),
            pltpu.VMEM((NCH, CH), jnp.int32),
            pltpu.SemaphoreType.DMA,
            pltpu.SemaphoreType.DMA,
        ],
    )
    def _dispatch_k(x_hbm, p0_hbm, p1_hbm, xd_hbm, rows_v, i0_v, i1_v, s0, s1):
        wid = lax.axis_index("s") * NC + lax.axis_index("c")
        pltpu.sync_copy(p0_hbm.at[wid], i0_v)
        pltpu.sync_copy(p1_hbm.at[wid], i1_v)
        for j in range(NCH):
            base = wid * TPW + j * CH
            pltpu.sync_copy(x_hbm.at[pl.ds(base, CH)], rows_v)
            c0 = pltpu.async_copy(rows_v, xd_hbm.at[i0_v.at[j]], s0)
            c1 = pltpu.async_copy(rows_v, xd_hbm.at[i1_v.at[j]], s1)
            c0.wait()
            c1.wait()

    return _dispatch_k


def _ffn_body(be_ref, xd_ref, wg_ref, wl_ref, bg_ref, bl_ref, w2_ref, b2_ref,
              o_ref):
    xb = xd_ref[...]
    hg = lax.dot_general(xb, wg_ref[0], (((1,), (1,)), ((), ())),
                         preferred_element_type=jnp.float32) + bg_ref[0]
    hl = lax.dot_general(xb, wl_ref[0], (((1,), (1,)), ((), ())),
                         preferred_element_type=jnp.float32) + bl_ref[0]
    hg = jnp.minimum(hg, LIMIT)
    hl = jnp.clip(hl, -LIMIT, LIMIT)
    y = hg * (1.0 / (1.0 + jnp.exp(-ALPHA * hg))) * (hl + 1.0)
    out = lax.dot_general(y, w2_ref[0], (((1,), (1,)), ((), ())),
                          preferred_element_type=jnp.float32)
    o_ref[...] = out + b2_ref[0]


def _ffn(be, xd, w1t, bg, bl, fc2_w, b2):
    return pl.pallas_call(
        _ffn_body,
        grid_spec=pltpu.PrefetchScalarGridSpec(
            num_scalar_prefetch=1,
            grid=(NBLK,),
            in_specs=[
                pl.BlockSpec((BLK, H), lambda i, be_r: (i, 0)),
                pl.BlockSpec((1, I, H), lambda i, be_r: (be_r[i], 0, 0)),
                pl.BlockSpec((1, I, H), lambda i, be_r: (be_r[i], 0, 1)),
                pl.BlockSpec((1, 1, I), lambda i, be_r: (be_r[i], 0, 0)),
                pl.BlockSpec((1, 1, I), lambda i, be_r: (be_r[i], 0, 0)),
                pl.BlockSpec((1, H, I), lambda i, be_r: (be_r[i], 0, 0)),
                pl.BlockSpec((1, 1, H), lambda i, be_r: (be_r[i], 0, 0)),
            ],
            out_specs=pl.BlockSpec((BLK, H), lambda i, be_r: (i, 0)),
        ),
        out_shape=jax.ShapeDtypeStruct((ND, H), jnp.float32),
    )(be, xd, w1t, w1t, bg, bl, fc2_w, b2)


@functools.cache
def _make_combine():
    mesh = plsc.VectorSubcoreMesh(core_axis_name="c", subcore_axis_name="s")

    @functools.partial(
        pl.kernel,
        mesh=mesh,
        out_type=jax.ShapeDtypeStruct((N, H), jnp.float32),
        scratch_types=[
            pltpu.VMEM((CH, H), jnp.float32),
            pltpu.VMEM((CH, H), jnp.float32),
            pltpu.VMEM((NCH, CH), jnp.int32),
            pltpu.VMEM((NCH, CH), jnp.int32),
            pltpu.VMEM((TPW, 16), jnp.float32),
            pltpu.VMEM((TPW, 16), jnp.float32),
            pltpu.SemaphoreType.DMA,
            pltpu.SemaphoreType.DMA,
        ],
    )
    def _combine_k(od_hbm, p0_hbm, p1_hbm, w0_hbm, w1_hbm, out_hbm,
                   a_v, b_v, i0_v, i1_v, wa_v, wb_v, sa, sb):
        wid = lax.axis_index("s") * NC + lax.axis_index("c")
        pltpu.sync_copy(p0_hbm.at[wid], i0_v)
        pltpu.sync_copy(p1_hbm.at[wid], i1_v)
        pltpu.sync_copy(w0_hbm.at[wid], wa_v)
        pltpu.sync_copy(w1_hbm.at[wid], wb_v)
        for j in range(NCH):
            ca = pltpu.async_copy(od_hbm.at[i0_v.at[j]], a_v, sa)
            cb = pltpu.async_copy(od_hbm.at[i1_v.at[j]], b_v, sb)
            ca.wait()
            cb.wait()

            def body_m(m, carry):
                wa = wa_v[j * CH + m, :]
                wb = wb_v[j * CH + m, :]
                for c in range(H // 16):
                    av = a_v[m, pl.ds(c * 16, 16)]
                    bv = b_v[m, pl.ds(c * 16, 16)]
                    a_v[m, pl.ds(c * 16, 16)] = av * wa + bv * wb
                return carry

            lax.fori_loop(0, CH, body_m, 0)
            pltpu.sync_copy(a_v, out_hbm.at[pl.ds(wid * TPW + j * CH, CH)])

    return _combine_k


def kernel(hidden_states, gate_w, gate_b, fc1_w, fc1_b, fc2_w, fc2_b):
    x = hidden_states.reshape(N, H)
    gb2 = jnp.broadcast_to(gate_b[None, :], (E, E))
    e0, e1, w0, w1 = _router(x, gate_w, gb2)

    eflat = jnp.concatenate([e0, e1], axis=1).reshape(64, 128)
    pos2d, bid = _perm(eflat)
    pos = pos2d.reshape(N, 2)
    p0 = pos[:, 0].reshape(NW, NCH, CH)
    p1 = pos[:, 1].reshape(NW, NCH, CH)
    be = bid.reshape(128)[:NBLK]

    xd = _make_dispatch()(x, p0, p1)

    bg = fc1_b[:, 0::2].reshape(E, 1, I)
    bl = fc1_b[:, 1::2].reshape(E, 1, I)
    w1r = fc1_w.reshape(E, I, 2 * H)
    od = _ffn(be, xd, w1r, bg, bl, fc2_w, fc2_b.reshape(E, 1, H))

    out = _make_combine()(od, p0, p1,
                          w0.reshape(NW, TPW, 16), w1.reshape(NW, TPW, 16))
    return out.reshape(B, S, H)
